# baseline (speedup 1.0000x reference)
import functools

import jax
import jax.numpy as jnp
from jax import lax
from jax.experimental import pallas as pl
from jax.experimental.pallas import tpu as pltpu
from jax.experimental.pallas import tpu_sc as plsc

_VOCAB_TILE = 1024


def _head_body(wtb_ref, h5_ref, out_ref):
    out_ref[...] = lax.dot_general(
        wtb_ref[...].astype(jnp.bfloat16),
        h5_ref[...].astype(jnp.bfloat16),
        dimension_numbers=(((0,), (0,)), ((), ())),
        preferred_element_type=jnp.float32,
    )


def _head(wtb, h5):
    V = wtb.shape[1]
    B = h5.shape[1]
    grid = pl.cdiv(V, _VOCAB_TILE)
    return pl.pallas_call(
        _head_body,
        grid=(grid,),
        in_specs=[
            pl.BlockSpec((5, _VOCAB_TILE), lambda j: (0, j)),
            pl.BlockSpec((5, B), lambda j: (0, 0)),
        ],
        out_specs=pl.BlockSpec((_VOCAB_TILE, B), lambda j: (j, 0)),
        out_shape=jax.ShapeDtypeStruct((V, B), jnp.float32),
    )(wtb, h5)


_LANES = 16


def _sc_gather(etab, ids, D, B):
    info = plsc.get_sparse_core_info()
    nw = info.num_cores * info.num_subcores
    bw = B // nw
    nrow = bw * D
    mesh = plsc.VectorSubcoreMesh(core_axis_name="c", subcore_axis_name="s")

    @functools.partial(
        pl.kernel,
        mesh=mesh,
        out_type=jax.ShapeDtypeStruct((B * D,), jnp.float32),
        scratch_types=[
            pltpu.VMEM((bw,), jnp.int32),
            pltpu.VMEM((nrow,), jnp.int32),
            pltpu.VMEM((nrow, 128), jnp.float32),
            pltpu.VMEM((nrow,), jnp.float32),
            pltpu.SemaphoreType.DMA,
        ],
        compiler_params=pltpu.CompilerParams(needs_layout_passes=False),
    )
    def k(etab_hbm, ids_hbm, out_hbm, idx_v, row_v, rows_v, out_v, sem):
        wid = lax.axis_index("s") * info.num_cores + lax.axis_index("c")
        base = wid * bw
        pltpu.sync_copy(ids_hbm.at[pl.ds(base, bw)], idx_v)
        for g in range(nrow // _LANES):
            n = lax.iota(jnp.int32, _LANES) + g * _LANES
            tok = lax.bitwise_and(n, bw - 1)
            kdim = lax.shift_right_logical(n, 5)
            tid = plsc.load_gather(idx_v, [tok])
            row_v[pl.ds(g * _LANES, _LANES)] = (
                lax.shift_left(lax.shift_right_logical(tid, 7), 3) + kdim
            )
        pltpu.async_copy(etab_hbm.at[row_v], rows_v, sem).wait()
        for g in range(nrow // _LANES):
            n = lax.iota(jnp.int32, _LANES) + g * _LANES
            tok = lax.bitwise_and(n, bw - 1)
            tid = plsc.load_gather(idx_v, [tok])
            col = lax.bitwise_and(tid, 127)
            out_v[pl.ds(g * _LANES, _LANES)] = plsc.load_gather(
                rows_v, [n, col]
            )
        for kk in range(D):
            pltpu.sync_copy(
                out_v.at[pl.ds(kk * bw, bw)],
                out_hbm.at[pl.ds(kk * B + base, bw)],
            )

    return k(etab, ids)


def kernel(input_ids, emb, W, b):
    B = input_ids.shape[0]
    V, D = emb.shape
    ids = input_ids.astype(jnp.int32)
    ntiles = (V + 127) // 128
    e8 = jnp.pad(emb.T, ((0, 8 - D), (0, ntiles * 128 - V)))
    etab = e8.reshape(8, ntiles, 128).transpose(1, 0, 2).reshape(-1, 128)
    h_t = _sc_gather(etab, ids, D, B).reshape(D, B)
    h5 = jnp.concatenate([h_t, jnp.ones((1, B), jnp.float32)], axis=0)
    wtb = jnp.concatenate([W.T, b.reshape(1, -1)], axis=0)
    out_t = _head(wtb, h5)
    return out_t.T


# speedup vs baseline: 1.0810x; 1.0810x over previous
import functools

import jax
import jax.numpy as jnp
from jax import lax
from jax.experimental import pallas as pl
from jax.experimental.pallas import tpu as pltpu
from jax.experimental.pallas import tpu_sc as plsc

_VOCAB_TILE = 2048


def _head_body(wtb_ref, h5_ref, out_ref):
    out_ref[...] = lax.dot_general(
        wtb_ref[...].astype(jnp.bfloat16),
        h5_ref[...].astype(jnp.bfloat16),
        dimension_numbers=(((0,), (0,)), ((), ())),
        preferred_element_type=jnp.float32,
    )


def _head(wtb, h5):
    V = wtb.shape[1]
    B = h5.shape[1]
    grid = pl.cdiv(V, _VOCAB_TILE)
    return pl.pallas_call(
        _head_body,
        grid=(grid,),
        in_specs=[
            pl.BlockSpec((5, _VOCAB_TILE), lambda j: (0, j)),
            pl.BlockSpec((5, B), lambda j: (0, 0)),
        ],
        out_specs=pl.BlockSpec((_VOCAB_TILE, B), lambda j: (j, 0)),
        out_shape=jax.ShapeDtypeStruct((V, B), jnp.float32),
    )(wtb, h5)


_LANES = 16


def _sc_gather(etab, ids, D, B):
    info = plsc.get_sparse_core_info()
    nw = info.num_cores * info.num_subcores
    bw = B // nw
    nrow = bw * D
    mesh = plsc.VectorSubcoreMesh(core_axis_name="c", subcore_axis_name="s")

    @functools.partial(
        pl.kernel,
        mesh=mesh,
        out_type=jax.ShapeDtypeStruct(((D + 1) * B,), jnp.float32),
        scratch_types=[
            pltpu.VMEM((bw,), jnp.int32),
            pltpu.VMEM((nrow,), jnp.int32),
            pltpu.VMEM((nrow, 128), jnp.float32),
            pltpu.VMEM((nrow,), jnp.float32),
            pltpu.VMEM((bw,), jnp.float32),
            pltpu.SemaphoreType.DMA,
        ],
        compiler_params=pltpu.CompilerParams(needs_layout_passes=False),
    )
    def k(etab_hbm, ids_hbm, out_hbm, idx_v, row_v, rows_v, out_v, one_v, sem):
        wid = lax.axis_index("s") * info.num_cores + lax.axis_index("c")
        base = wid * bw
        pltpu.sync_copy(ids_hbm.at[pl.ds(base, bw)], idx_v)
        for g in range(nrow // _LANES):
            n = lax.iota(jnp.int32, _LANES) + g * _LANES
            tok = lax.bitwise_and(n, bw - 1)
            kdim = lax.shift_right_logical(n, 5)
            tid = plsc.load_gather(idx_v, [tok])
            row_v[pl.ds(g * _LANES, _LANES)] = (
                lax.shift_left(lax.shift_right_logical(tid, 7), 3) + kdim
            )
        pltpu.async_copy(etab_hbm.at[row_v], rows_v, sem).wait()
        for g in range(nrow // _LANES):
            n = lax.iota(jnp.int32, _LANES) + g * _LANES
            tok = lax.bitwise_and(n, bw - 1)
            tid = plsc.load_gather(idx_v, [tok])
            col = lax.bitwise_and(tid, 127)
            out_v[pl.ds(g * _LANES, _LANES)] = plsc.load_gather(
                rows_v, [n, col]
            )
        for kk in range(D):
            pltpu.sync_copy(
                out_v.at[pl.ds(kk * bw, bw)],
                out_hbm.at[pl.ds(kk * B + base, bw)],
            )
        for c in range(bw // _LANES):
            one_v[pl.ds(c * _LANES, _LANES)] = jnp.full(
                (_LANES,), 1.0, jnp.float32
            )
        pltpu.sync_copy(one_v, out_hbm.at[pl.ds(D * B + base, bw)])

    return k(etab, ids)


def kernel(input_ids, emb, W, b):
    B = input_ids.shape[0]
    V, D = emb.shape
    ids = input_ids.astype(jnp.int32)
    ntiles = (V + 127) // 128
    e8 = jnp.pad(emb.T, ((0, 8 - D), (0, ntiles * 128 - V)))
    etab = e8.reshape(8, ntiles, 128).transpose(1, 0, 2).reshape(-1, 128)
    h5 = _sc_gather(etab, ids, D, B).reshape(D + 1, B)
    wtb = jnp.concatenate([W.T, b.reshape(1, -1)], axis=0)
    out_t = _head(wtb, h5)
    return out_t.T
